# baseline (speedup 1.0000x reference)
import functools

import jax
import jax.numpy as jnp
from jax import lax
from jax.experimental import pallas as pl
from jax.experimental.pallas import tpu as pltpu
from jax.experimental.pallas import tpu_sc as plsc

WORD_DIM = 300
POS_DIM = 50
OUT_DIM = WORD_DIM + POS_DIM
WPAD = 384
TILE = 128
PSHIFT = WORD_DIM - 2 * TILE
NUM_CORES = 2
NUM_SUBCORES = 16
NUM_WORKERS = NUM_CORES * NUM_SUBCORES
CHUNK = 64


def kernel(tokens, tags, word_table, pos_table):
    B, L = tokens.shape
    N = B * L
    n_per_w = N // NUM_WORKERS
    n_chunks = n_per_w // CHUNK

    tok = tokens.reshape(N).astype(jnp.int32)
    tag = tags.reshape(N).astype(jnp.int32)
    wt0 = word_table[:, :TILE]
    wt1 = word_table[:, TILE:2 * TILE]
    wt2 = jnp.pad(word_table[:, 2 * TILE:], ((0, 0), (0, WPAD - WORD_DIM)))
    ppad = jnp.pad(pos_table,
                   ((0, 0), (PSHIFT, TILE - POS_DIM - PSHIFT)))

    mesh = plsc.VectorSubcoreMesh(core_axis_name="c", subcore_axis_name="s")

    buf_types = []
    for _ in range(2):
        buf_types += [
            pltpu.VMEM((CHUNK,), jnp.int32),
            pltpu.VMEM((CHUNK,), jnp.int32),
            pltpu.VMEM((CHUNK, TILE), jnp.float32),
            pltpu.VMEM((CHUNK, TILE), jnp.float32),
            pltpu.VMEM((CHUNK, TILE), jnp.float32),
            pltpu.VMEM((CHUNK, TILE), jnp.float32),
            pltpu.SemaphoreType.DMA,
            pltpu.SemaphoreType.DMA,
            pltpu.SemaphoreType.DMA,
            pltpu.SemaphoreType.DMA,
            pltpu.SemaphoreType.DMA,
        ]

    @functools.partial(
        pl.kernel,
        mesh=mesh,
        out_type=jax.ShapeDtypeStruct((N, WPAD), jnp.float32),
        scratch_types=buf_types,
    )
    def gather_sc(tok_hbm, tag_hbm, w0_hbm, w1_hbm, w2_hbm, pos_hbm, out_hbm,
                  *bufs):
        wid = lax.axis_index("s") * NUM_CORES + lax.axis_index("c")
        wbase = wid * n_per_w
        a, b = bufs[:11], bufs[11:]

        def prefetch(j, buf):
            tok_v, tag_v, v0, v1, v2, pv, s0, s1, s2, sp, _ = buf
            base = wbase + j * CHUNK
            pltpu.sync_copy(tok_hbm.at[pl.ds(base, CHUNK)], tok_v)
            pltpu.sync_copy(tag_hbm.at[pl.ds(base, CHUNK)], tag_v)
            pltpu.async_copy(w0_hbm.at[tok_v], v0, s0)
            pltpu.async_copy(w1_hbm.at[tok_v], v1, s1)
            pltpu.async_copy(w2_hbm.at[tok_v], v2, s2)
            pltpu.async_copy(pos_hbm.at[tag_v], pv, sp)

        def finish(j, buf):
            tok_v, tag_v, v0, v1, v2, pv, s0, s1, s2, sp, so = buf
            base = wbase + j * CHUNK
            pltpu.make_async_copy(w0_hbm.at[tok_v], v0, s0).wait()
            pltpu.make_async_copy(w1_hbm.at[tok_v], v1, s1).wait()
            pltpu.make_async_copy(w2_hbm.at[tok_v], v2, s2).wait()
            pltpu.make_async_copy(pos_hbm.at[tag_v], pv, sp).wait()

            @pl.loop(0, CHUNK)
            def _(r):
                for k in range(TILE // 16):
                    v2[r, pl.ds(16 * k, 16)] = (
                        v2[r, pl.ds(16 * k, 16)] + pv[r, pl.ds(16 * k, 16)])

            rows = out_hbm.at[pl.ds(base, CHUNK)]
            pltpu.async_copy(v0, rows.at[:, pl.ds(0, TILE)], so)
            pltpu.async_copy(v1, rows.at[:, pl.ds(TILE, TILE)], so)
            pltpu.async_copy(v2, rows.at[:, pl.ds(2 * TILE, TILE)], so)

        def wait_out(j, buf):
            _, _, v0, v1, v2, _, _, _, _, _, so = buf
            base = wbase + j * CHUNK
            rows = out_hbm.at[pl.ds(base, CHUNK)]
            pltpu.make_async_copy(v0, rows.at[:, pl.ds(0, TILE)], so).wait()
            pltpu.make_async_copy(v1, rows.at[:, pl.ds(TILE, TILE)], so).wait()
            pltpu.make_async_copy(v2, rows.at[:, pl.ds(2 * TILE, TILE)], so).wait()

        prefetch(0, a)

        @pl.loop(0, n_chunks // 2)
        def _(m):
            j0 = 2 * m

            @pl.when(m > 0)
            def _():
                wait_out(j0 - 1, b)

            prefetch(j0 + 1, b)
            finish(j0, a)
            wait_out(j0, a)

            @pl.when(j0 + 2 < n_chunks)
            def _():
                prefetch(j0 + 2, a)

            finish(j0 + 1, b)

        wait_out(n_chunks - 1, b)

    out = gather_sc(tok, tag, wt0, wt1, wt2, ppad)
    return out[:, :OUT_DIM].reshape(B, L, OUT_DIM)


# speedup vs baseline: 1.2215x; 1.2215x over previous
import functools

import jax
import jax.numpy as jnp
from jax import lax
from jax.experimental import pallas as pl
from jax.experimental.pallas import tpu as pltpu
from jax.experimental.pallas import tpu_sc as plsc

WORD_DIM = 300
POS_DIM = 50
OUT_DIM = WORD_DIM + POS_DIM
WPAD = 384
TILE = 128
PSHIFT = WORD_DIM - 2 * TILE
NUM_CORES = 2
NUM_SUBCORES = 16
NUM_WORKERS = NUM_CORES * NUM_SUBCORES
CHUNK = 128
NTAG_PAD = 56


def kernel(tokens, tags, word_table, pos_table):
    B, L = tokens.shape
    N = B * L
    n_per_w = N // NUM_WORKERS
    n_chunks = n_per_w // CHUNK

    tok = tokens.reshape(N).astype(jnp.int32)
    tag = tags.reshape(N).astype(jnp.int32)
    wt2 = jnp.pad(word_table[:, 2 * TILE:], ((0, 0), (0, WPAD - WORD_DIM)))
    ppad = jnp.pad(pos_table,
                   ((0, NTAG_PAD - pos_table.shape[0]),
                    (PSHIFT, TILE - POS_DIM - PSHIFT)))

    mesh = plsc.VectorSubcoreMesh(core_axis_name="c", subcore_axis_name="s")

    @functools.partial(
        pl.kernel,
        mesh=mesh,
        compiler_params=pltpu.CompilerParams(needs_layout_passes=False),
        out_type=jax.ShapeDtypeStruct((N, WPAD), jnp.float32),
        scratch_types=[
            pltpu.VMEM((CHUNK,), jnp.int32),
            pltpu.VMEM((CHUNK,), jnp.int32),
            pltpu.VMEM((CHUNK, TILE), jnp.float32),
            pltpu.VMEM((CHUNK, TILE), jnp.float32),
            pltpu.VMEM((CHUNK, TILE), jnp.float32),
            pltpu.VMEM((NTAG_PAD, TILE), jnp.float32),
            pltpu.SemaphoreType.DMA,
            pltpu.SemaphoreType.DMA,
            pltpu.SemaphoreType.DMA,
        ],
    )
    def gather_sc(tok_hbm, tag_hbm, word_hbm, w2_hbm, pos_hbm, out_hbm,
                  tok_v, tag_v, v0, v1, v2, ptab, s0, s1, s2):
        wid = lax.axis_index("s") * NUM_CORES + lax.axis_index("c")
        wbase = wid * n_per_w
        w0_hbm = word_hbm.at[:, pl.ds(0, TILE)]
        w1_hbm = word_hbm.at[:, pl.ds(TILE, TILE)]

        pltpu.sync_copy(pos_hbm, ptab)
        cols = [lax.iota(jnp.int32, 16) + 16 * k for k in range(2, 6)]

        @pl.loop(0, n_chunks)
        def _(i):
            base = wbase + i * CHUNK
            pltpu.sync_copy(tok_hbm.at[pl.ds(base, CHUNK)], tok_v)
            pltpu.sync_copy(tag_hbm.at[pl.ds(base, CHUNK)], tag_v)
            c0 = pltpu.async_copy(w0_hbm.at[tok_v], v0, s0)
            c1 = pltpu.async_copy(w1_hbm.at[tok_v], v1, s1)
            c2 = pltpu.async_copy(w2_hbm.at[tok_v], v2, s2)
            c0.wait()
            c1.wait()
            c2.wait()

            @pl.loop(0, CHUNK)
            def _(r):
                t16 = plsc.load_gather(tag_v, [jnp.zeros((16,), jnp.int32) + r])
                for k in range(4):
                    prow = plsc.load_gather(ptab, [t16, cols[k]])
                    c = 32 + 16 * k
                    v2[r, pl.ds(c, 16)] = v2[r, pl.ds(c, 16)] + prow

            rows = out_hbm.at[pl.ds(base, CHUNK)]
            pltpu.sync_copy(v0, rows.at[:, pl.ds(0, TILE)])
            pltpu.sync_copy(v1, rows.at[:, pl.ds(TILE, TILE)])
            pltpu.sync_copy(v2, rows.at[:, pl.ds(2 * TILE, TILE)])

    out = gather_sc(tok, tag, word_table, wt2, ppad)
    return out[:, :OUT_DIM].reshape(B, L, OUT_DIM)


# speedup vs baseline: 1.2888x; 1.0551x over previous
import functools

import jax
import jax.numpy as jnp
from jax import lax
from jax.experimental import pallas as pl
from jax.experimental.pallas import tpu as pltpu
from jax.experimental.pallas import tpu_sc as plsc

WORD_DIM = 300
POS_DIM = 50
OUT_DIM = WORD_DIM + POS_DIM
WPAD = 384
TILE = 128
PSHIFT = WORD_DIM - 2 * TILE
NUM_CORES = 2
NUM_SUBCORES = 16
NUM_WORKERS = NUM_CORES * NUM_SUBCORES
CHUNK = 128
NTAG_PAD = 56


def kernel(tokens, tags, word_table, pos_table):
    B, L = tokens.shape
    N = B * L
    n_per_w = N // NUM_WORKERS
    n_chunks = n_per_w // CHUNK

    tok = tokens.reshape(N).astype(jnp.int32)
    tag = tags.reshape(N).astype(jnp.int32)
    wt2 = jnp.pad(word_table[:, 2 * TILE:], ((0, 0), (0, WPAD - WORD_DIM)))
    ppad = jnp.pad(pos_table,
                   ((0, NTAG_PAD - pos_table.shape[0]),
                    (PSHIFT, TILE - POS_DIM - PSHIFT)))

    mesh = plsc.VectorSubcoreMesh(core_axis_name="c", subcore_axis_name="s")

    @functools.partial(
        pl.kernel,
        mesh=mesh,
        compiler_params=pltpu.CompilerParams(needs_layout_passes=False),
        out_type=jax.ShapeDtypeStruct((N, WPAD), jnp.float32),
        scratch_types=[
            pltpu.VMEM((CHUNK,), jnp.int32),
            pltpu.VMEM((CHUNK,), jnp.int32),
            pltpu.VMEM((CHUNK, TILE), jnp.float32),
            pltpu.VMEM((CHUNK, TILE), jnp.float32),
            pltpu.VMEM((CHUNK, TILE), jnp.float32),
            pltpu.VMEM((NTAG_PAD, TILE), jnp.float32),
            pltpu.SemaphoreType.DMA,
            pltpu.SemaphoreType.DMA,
            pltpu.SemaphoreType.DMA,
            pltpu.SemaphoreType.DMA,
        ],
    )
    def gather_sc(tok_hbm, tag_hbm, word_hbm, w2_hbm, pos_hbm, out_hbm,
                  tok_v, tag_v, v0, v1, v2, ptab, s0, s1, s2, so):
        wid = lax.axis_index("s") * NUM_CORES + lax.axis_index("c")
        wbase = wid * n_per_w
        w0_hbm = word_hbm.at[:, pl.ds(0, TILE)]
        w1_hbm = word_hbm.at[:, pl.ds(TILE, TILE)]

        pltpu.sync_copy(pos_hbm, ptab)
        cols = [lax.iota(jnp.int32, 16) + 16 * k for k in range(2, 6)]

        @pl.loop(0, n_chunks)
        def _(i):
            base = wbase + i * CHUNK
            pltpu.sync_copy(tok_hbm.at[pl.ds(base, CHUNK)], tok_v)
            pltpu.sync_copy(tag_hbm.at[pl.ds(base, CHUNK)], tag_v)
            c0 = pltpu.async_copy(w0_hbm.at[tok_v], v0, s0)
            c1 = pltpu.async_copy(w1_hbm.at[tok_v], v1, s1)
            c2 = pltpu.async_copy(w2_hbm.at[tok_v], v2, s2)
            rows = out_hbm.at[pl.ds(base, CHUNK)]
            c0.wait()
            c1.wait()
            o0 = pltpu.async_copy(v0, rows.at[:, pl.ds(0, TILE)], so)
            o1 = pltpu.async_copy(v1, rows.at[:, pl.ds(TILE, TILE)], so)
            c2.wait()

            @pl.loop(0, CHUNK)
            def _(r):
                t16 = plsc.load_gather(tag_v, [jnp.zeros((16,), jnp.int32) + r])
                for k in range(4):
                    prow = plsc.load_gather(ptab, [t16, cols[k]])
                    c = 32 + 16 * k
                    v2[r, pl.ds(c, 16)] = v2[r, pl.ds(c, 16)] + prow

            o2 = pltpu.async_copy(v2, rows.at[:, pl.ds(2 * TILE, TILE)], so)
            o0.wait()
            o1.wait()
            o2.wait()

    out = gather_sc(tok, tag, word_table, wt2, ppad)
    return out[:, :OUT_DIM].reshape(B, L, OUT_DIM)
